# baseline (speedup 1.0000x reference)
import functools

import jax
import jax.numpy as jnp
from jax import lax
from jax.experimental import pallas as pl
from jax.experimental.pallas import tpu as pltpu
from jax.experimental.pallas import tpu_sc as plsc

NUM_CORES = 2
NUM_SUBCORES = 16
NUM_TILES = NUM_CORES * NUM_SUBCORES
LANES = 16

H = 4
C = 32
HC = H * C
PADW = HC + 16
CHUNK = 64
BLK = 40


def _matmul_body(x_ref, w_ref, o_ref):
    o_ref[...] = jnp.dot(x_ref[...], w_ref[...],
                         preferred_element_type=jnp.float32)


def _project(x, W):
    n, in_dim = x.shape
    bm = 1000
    grid = n // bm
    return pl.pallas_call(
        _matmul_body,
        grid=(grid,),
        in_specs=[
            pl.BlockSpec((bm, in_dim), lambda i: (i, 0)),
            pl.BlockSpec((in_dim, HC), lambda i: (0, 0)),
        ],
        out_specs=pl.BlockSpec((bm, HC), lambda i: (i, 0)),
        out_shape=jax.ShapeDtypeStruct((n, HC), jnp.float32),
    )(x, W)


def _sc_edge_body(n_nodes, n_edges, proj, rows_in, cols_in, att_in, out,
                  row_idx, col_idx, src_buf, dst_buf, w_buf, att_v, shared,
                  sem):
    cid = lax.axis_index("c")
    sid = lax.axis_index("s")
    wid = sid * NUM_CORES + cid

    pltpu.sync_copy(att_in, att_v)

    zeros16 = jnp.zeros((LANES,), jnp.float32)

    @pl.loop(0, CHUNK)
    def _zero_w(r):
        for cb in range(PADW // LANES):
            w_buf[r, pl.ds(cb * LANES, LANES)] = zeros16

    n_blocks = n_nodes // BLK
    for k in range((n_blocks + NUM_SUBCORES - 1) // NUM_SUBCORES):
        b = sid + k * NUM_SUBCORES

        @pl.when(b < n_blocks)
        def _():
            pltpu.sync_copy(w_buf.at[pl.ds(0, BLK)],
                            shared.at[pl.ds(b * BLK, BLK)])
    plsc.subcore_barrier()

    lane_iota = lax.iota(jnp.int32, LANES)
    VPR = HC // LANES
    att_blk = [att_v[pl.ds(b * LANES, LANES)] for b in range(VPR)]
    onehot = [(lane_iota == h).astype(jnp.float32) for h in range(H)]

    n_chunks = n_edges // CHUNK
    iters = (n_chunks + NUM_TILES - 1) // NUM_TILES

    @pl.loop(0, iters)
    def _chunk(i):
        c = wid + i * NUM_TILES

        @pl.when(c < n_chunks)
        def _do_chunk():
            ebase = c * CHUNK
            pltpu.sync_copy(rows_in.at[pl.ds(ebase, CHUNK)], row_idx)
            pltpu.sync_copy(cols_in.at[pl.ds(ebase, CHUNK)], col_idx)
            g1 = pltpu.async_copy(proj.at[row_idx], src_buf, sem)
            g2 = pltpu.async_copy(proj.at[col_idx], dst_buf, sem)
            g1.wait()
            g2.wait()

            @pl.loop(0, CHUNK, unroll=2)
            def _edge(e):
                sv = [src_buf[e, pl.ds(k * LANES, LANES)]
                      for k in range(VPR)]
                w = []
                for k in range(VPR):
                    dv = dst_buf[e, pl.ds(k * LANES, LANES)]
                    u = sv[k] + dv
                    e2 = jnp.exp(u + u)
                    t = (e2 - 1.0) / (e2 + 1.0)
                    w.append(t * att_blk[k])
                nvec = jnp.zeros((LANES,), jnp.float32)
                sh = []
                for h in range(H):
                    raw = jnp.sum(w[2 * h] + w[2 * h + 1])
                    eh = jnp.exp(jnp.full((LANES,), raw, jnp.float32))
                    sh.append(eh)
                    nvec = nvec + eh * onehot[h]
                for k in range(VPR):
                    w_buf[e, pl.ds(k * LANES, LANES)] = sv[k] * sh[k // 2]
                w_buf[e, pl.ds(HC, LANES)] = nvec

            pltpu.sync_copy(w_buf, shared.at[col_idx], add=True)

    plsc.subcore_barrier()

    for k in range((n_blocks + NUM_SUBCORES - 1) // NUM_SUBCORES):
        b = sid + k * NUM_SUBCORES

        @pl.when(b < n_blocks)
        def _():
            pltpu.sync_copy(shared.at[pl.ds(b * BLK, BLK)],
                            w_buf.at[pl.ds(0, BLK)])
            pltpu.sync_copy(w_buf.at[pl.ds(0, BLK)],
                            out.at[cid, pl.ds(b * BLK, BLK)])


def _sc_edge_pass(proj, rows, cols, att_flat):
    n = proj.shape[0]
    e = rows.shape[0]
    mesh = plsc.VectorSubcoreMesh(core_axis_name="c", subcore_axis_name="s")
    return pl.kernel(
        functools.partial(_sc_edge_body, n, e),
        out_type=jax.ShapeDtypeStruct((NUM_CORES, n, PADW), jnp.float32),
        mesh=mesh,
        compiler_params=pltpu.CompilerParams(use_tc_tiling_on_sc=False,
                                             needs_layout_passes=False),
        scratch_types=[
            pltpu.VMEM((CHUNK,), jnp.int32),
            pltpu.VMEM((CHUNK,), jnp.int32),
            pltpu.VMEM((CHUNK, HC), jnp.float32),
            pltpu.VMEM((CHUNK, HC), jnp.float32),
            pltpu.VMEM((CHUNK, PADW), jnp.float32),
            pltpu.VMEM((HC,), jnp.float32),
            pltpu.VMEM_SHARED((n, PADW), jnp.float32),
            pltpu.SemaphoreType.DMA,
        ],
    )(proj, rows, cols, att_flat)


def _finalize_body(p_ref, o_ref):
    p = p_ref[0] + p_ref[1]
    w = p[:, :HC]
    nrm = jnp.maximum(p[:, HC:HC + H], 1e-12)
    o_ref[...] = w / jnp.repeat(nrm, C, axis=1)


def _finalize(partials):
    n = partials.shape[1]
    bm = 1000
    return pl.pallas_call(
        _finalize_body,
        grid=(n // bm,),
        in_specs=[pl.BlockSpec((NUM_CORES, bm, PADW), lambda i: (0, i, 0))],
        out_specs=pl.BlockSpec((bm, HC), lambda i: (i, 0)),
        out_shape=jax.ShapeDtypeStruct((n, HC), jnp.float32),
    )(partials)


def kernel(x, edge_index, W, att):
    proj = _project(x, W)
    partials = _sc_edge_pass(proj, edge_index[0], edge_index[1],
                             att.reshape(-1))
    return _finalize(partials)


# speedup vs baseline: 1.1854x; 1.1854x over previous
import functools

import jax
import jax.numpy as jnp
from jax import lax
from jax.experimental import pallas as pl
from jax.experimental.pallas import tpu as pltpu
from jax.experimental.pallas import tpu_sc as plsc

NUM_CORES = 2
NUM_SUBCORES = 16
NUM_TILES = NUM_CORES * NUM_SUBCORES
LANES = 16

H = 4
C = 32
HC = H * C
PADW = HC + 16
CHUNK = 40
BLK = 40


def _matmul_body(x_ref, w_ref, o_ref):
    o_ref[...] = jnp.dot(x_ref[...], w_ref[...],
                         preferred_element_type=jnp.float32)


def _project(x, W):
    n, in_dim = x.shape
    bm = 1000
    grid = n // bm
    return pl.pallas_call(
        _matmul_body,
        grid=(grid,),
        in_specs=[
            pl.BlockSpec((bm, in_dim), lambda i: (i, 0)),
            pl.BlockSpec((in_dim, HC), lambda i: (0, 0)),
        ],
        out_specs=pl.BlockSpec((bm, HC), lambda i: (i, 0)),
        out_shape=jax.ShapeDtypeStruct((n, HC), jnp.float32),
    )(x, W)


def _sc_edge_body(n_nodes, n_edges, proj, rows_in, cols_in, att_in, out,
                  row_idx, col_idx, src_buf, dst_buf,
                  row_idx2, col_idx2, src_buf2, dst_buf2,
                  w_buf, att_v, shared, sem, sem2):
    cid = lax.axis_index("c")
    sid = lax.axis_index("s")
    wid = sid * NUM_CORES + cid

    pltpu.sync_copy(att_in, att_v)

    zeros16 = jnp.zeros((LANES,), jnp.float32)

    @pl.loop(0, CHUNK)
    def _zero_w(r):
        for cb in range(PADW // LANES):
            w_buf[r, pl.ds(cb * LANES, LANES)] = zeros16

    n_blocks = n_nodes // BLK
    for k in range((n_blocks + NUM_SUBCORES - 1) // NUM_SUBCORES):
        b = sid + k * NUM_SUBCORES

        @pl.when(b < n_blocks)
        def _():
            pltpu.sync_copy(w_buf.at[pl.ds(0, BLK)],
                            shared.at[pl.ds(b * BLK, BLK)])
    plsc.subcore_barrier()

    lane_iota = lax.iota(jnp.int32, LANES)
    VPR = HC // LANES
    att_blk = [att_v[pl.ds(b * LANES, LANES)] for b in range(VPR)]
    onehot = [(lane_iota == h).astype(jnp.float32) for h in range(H)]

    n_chunks = n_edges // CHUNK

    def _stage(c, r_idx, c_idx, s_buf, d_buf, s_sem):
        ebase = c * CHUNK
        pltpu.sync_copy(rows_in.at[pl.ds(ebase, CHUNK)], r_idx)
        pltpu.sync_copy(cols_in.at[pl.ds(ebase, CHUNK)], c_idx)
        g1 = pltpu.async_copy(proj.at[r_idx], s_buf, s_sem)
        g2 = pltpu.async_copy(proj.at[c_idx], d_buf, s_sem)
        return g1, g2

    def _compute(c_idx, s_buf, d_buf):
        @pl.loop(0, CHUNK)
        def _edge(e):
            sv = [s_buf[e, pl.ds(k * LANES, LANES)] for k in range(VPR)]
            w = []
            for k in range(VPR):
                dv = d_buf[e, pl.ds(k * LANES, LANES)]
                u = sv[k] + dv
                e2 = jnp.exp(u + u)
                t = (e2 - 1.0) / (e2 + 1.0)
                w.append(t * att_blk[k])
            nvec = jnp.zeros((LANES,), jnp.float32)
            sh = []
            for h in range(H):
                raw = jnp.sum(w[2 * h] + w[2 * h + 1])
                eh = jnp.exp(jnp.full((LANES,), raw, jnp.float32))
                sh.append(eh)
                nvec = nvec + eh * onehot[h]
            for k in range(VPR):
                w_buf[e, pl.ds(k * LANES, LANES)] = sv[k] * sh[k // 2]
            w_buf[e, pl.ds(HC, LANES)] = nvec

        pltpu.sync_copy(w_buf, shared.at[c_idx], add=True)

    ga = _stage(wid, row_idx, col_idx, src_buf, dst_buf, sem)

    @pl.loop(0, n_chunks // NUM_TILES // 2)
    def _chunk(i):
        ca = wid + (2 * i) * NUM_TILES
        cb = ca + NUM_TILES
        gb = _stage(cb, row_idx2, col_idx2, src_buf2, dst_buf2, sem2)
        pltpu.make_async_copy(proj.at[row_idx], src_buf, sem).wait()
        pltpu.make_async_copy(proj.at[col_idx], dst_buf, sem).wait()
        _compute(col_idx, src_buf, dst_buf)

        @pl.when(cb + NUM_TILES < n_chunks)
        def _():
            _stage(cb + NUM_TILES, row_idx, col_idx, src_buf, dst_buf, sem)
        gb[0].wait()
        gb[1].wait()
        _compute(col_idx2, src_buf2, dst_buf2)

    plsc.subcore_barrier()

    for k in range((n_blocks + NUM_SUBCORES - 1) // NUM_SUBCORES):
        b = sid + k * NUM_SUBCORES

        @pl.when(b < n_blocks)
        def _():
            pltpu.sync_copy(shared.at[pl.ds(b * BLK, BLK)],
                            w_buf.at[pl.ds(0, BLK)])
            pltpu.sync_copy(w_buf.at[pl.ds(0, BLK)],
                            out.at[cid, pl.ds(b * BLK, BLK)])


def _sc_edge_pass(proj, rows, cols, att_flat):
    n = proj.shape[0]
    e = rows.shape[0]
    mesh = plsc.VectorSubcoreMesh(core_axis_name="c", subcore_axis_name="s")
    return pl.kernel(
        functools.partial(_sc_edge_body, n, e),
        out_type=jax.ShapeDtypeStruct((NUM_CORES, n, PADW), jnp.float32),
        mesh=mesh,
        compiler_params=pltpu.CompilerParams(use_tc_tiling_on_sc=False,
                                             needs_layout_passes=False),
        scratch_types=[
            pltpu.VMEM((CHUNK,), jnp.int32),
            pltpu.VMEM((CHUNK,), jnp.int32),
            pltpu.VMEM((CHUNK, HC), jnp.float32),
            pltpu.VMEM((CHUNK, HC), jnp.float32),
            pltpu.VMEM((CHUNK,), jnp.int32),
            pltpu.VMEM((CHUNK,), jnp.int32),
            pltpu.VMEM((CHUNK, HC), jnp.float32),
            pltpu.VMEM((CHUNK, HC), jnp.float32),
            pltpu.VMEM((CHUNK, PADW), jnp.float32),
            pltpu.VMEM((HC,), jnp.float32),
            pltpu.VMEM_SHARED((n, PADW), jnp.float32),
            pltpu.SemaphoreType.DMA,
            pltpu.SemaphoreType.DMA,
        ],
    )(proj, rows, cols, att_flat)


def _finalize_body(p_ref, o_ref):
    p = p_ref[0] + p_ref[1]
    w = p[:, :HC]
    nrm = jnp.maximum(p[:, HC:HC + H], 1e-12)
    o_ref[...] = w / jnp.repeat(nrm, C, axis=1)


def _finalize(partials):
    n = partials.shape[1]
    bm = 1000
    return pl.pallas_call(
        _finalize_body,
        grid=(n // bm,),
        in_specs=[pl.BlockSpec((NUM_CORES, bm, PADW), lambda i: (0, i, 0))],
        out_specs=pl.BlockSpec((bm, HC), lambda i: (i, 0)),
        out_shape=jax.ShapeDtypeStruct((n, HC), jnp.float32),
    )(partials)


def kernel(x, edge_index, W, att):
    proj = _project(x, W)
    partials = _sc_edge_pass(proj, edge_index[0], edge_index[1],
                             att.reshape(-1))
    return _finalize(partials)


# speedup vs baseline: 1.4335x; 1.2093x over previous
import functools

import jax
import jax.numpy as jnp
from jax import lax
from jax.experimental import pallas as pl
from jax.experimental.pallas import tpu as pltpu
from jax.experimental.pallas import tpu_sc as plsc

NUM_CORES = 2
NUM_SUBCORES = 16
NUM_TILES = NUM_CORES * NUM_SUBCORES
LANES = 16

H = 4
C = 32
HC = H * C
PADW = HC + 16
CHUNK = 40
IBC = 10
BLK = 40


def _matmul_body(x_ref, w_ref, o_ref):
    o_ref[...] = jnp.dot(x_ref[...], w_ref[...],
                         preferred_element_type=jnp.float32)


def _project(x, W):
    n, in_dim = x.shape
    bm = 1000
    grid = n // bm
    return pl.pallas_call(
        _matmul_body,
        grid=(grid,),
        in_specs=[
            pl.BlockSpec((bm, in_dim), lambda i: (i, 0)),
            pl.BlockSpec((in_dim, HC), lambda i: (0, 0)),
        ],
        out_specs=pl.BlockSpec((bm, HC), lambda i: (i, 0)),
        out_shape=jax.ShapeDtypeStruct((n, HC), jnp.float32),
    )(x, W)


def _sc_edge_body(n_nodes, n_edges, proj, rows_in, cols_in, att_in, out,
                  row_blk, col_blk, src_buf, dst_buf, src_buf2, dst_buf2,
                  w_buf, att_v, shared, sem, sem2):
    cid = lax.axis_index("c")
    sid = lax.axis_index("s")
    wid = sid * NUM_CORES + cid

    pltpu.sync_copy(att_in, att_v)

    zeros16 = jnp.zeros((LANES,), jnp.float32)

    @pl.loop(0, CHUNK)
    def _zero_w(r):
        for cb in range(PADW // LANES):
            w_buf[r, pl.ds(cb * LANES, LANES)] = zeros16

    n_blocks = n_nodes // BLK
    for k in range((n_blocks + NUM_SUBCORES - 1) // NUM_SUBCORES):
        b = sid + k * NUM_SUBCORES

        @pl.when(b < n_blocks)
        def _():
            pltpu.sync_copy(w_buf.at[pl.ds(0, BLK)],
                            shared.at[pl.ds(b * BLK, BLK)])
    plsc.subcore_barrier()

    lane_iota = lax.iota(jnp.int32, LANES)
    VPR = HC // LANES
    att_blk = [att_v[pl.ds(b * LANES, LANES)] for b in range(VPR)]
    onehot = [(lane_iota == h).astype(jnp.float32) for h in range(H)]

    n_chunks = n_edges // CHUNK
    per_tile_chunks = n_chunks // NUM_TILES
    n_iblocks = per_tile_chunks // IBC

    def _fire(j, s_buf, d_buf, s_sem):
        g1 = pltpu.async_copy(proj.at[row_blk.at[j]], s_buf, s_sem)
        g2 = pltpu.async_copy(proj.at[col_blk.at[j]], d_buf, s_sem)
        return g1, g2

    def _compute(c_idx, s_buf, d_buf):
        @pl.loop(0, CHUNK)
        def _edge(e):
            sv = [s_buf[e, pl.ds(k * LANES, LANES)] for k in range(VPR)]
            w = []
            for k in range(VPR):
                dv = d_buf[e, pl.ds(k * LANES, LANES)]
                u = sv[k] + dv
                e2 = jnp.exp(u + u)
                t = (e2 - 1.0) / (e2 + 1.0)
                w.append(t * att_blk[k])
            nvec = jnp.zeros((LANES,), jnp.float32)
            sh = []
            for h in range(H):
                raw = jnp.sum(w[2 * h] + w[2 * h + 1])
                eh = jnp.exp(jnp.full((LANES,), raw, jnp.float32))
                sh.append(eh)
                nvec = nvec + eh * onehot[h]
            for k in range(VPR):
                w_buf[e, pl.ds(k * LANES, LANES)] = sv[k] * sh[k // 2]
            w_buf[e, pl.ds(HC, LANES)] = nvec

        pltpu.sync_copy(w_buf, shared.at[c_idx], add=True)

    @pl.loop(0, n_iblocks)
    def _iblock(b):
        rbase = wid * per_tile_chunks + b * IBC
        pltpu.sync_copy(rows_in.at[pl.ds(rbase, IBC)], row_blk)
        pltpu.sync_copy(cols_in.at[pl.ds(rbase, IBC)], col_blk)
        _fire(0, src_buf, dst_buf, sem)

        @pl.loop(0, IBC // 2)
        def _pair(j):
            gb = _fire(2 * j + 1, src_buf2, dst_buf2, sem2)
            pltpu.make_async_copy(proj.at[row_blk.at[2 * j]],
                                  src_buf, sem).wait()
            pltpu.make_async_copy(proj.at[col_blk.at[2 * j]],
                                  dst_buf, sem).wait()
            _compute(col_blk.at[2 * j], src_buf, dst_buf)

            @pl.when(j < IBC // 2 - 1)
            def _():
                _fire(2 * j + 2, src_buf, dst_buf, sem)
            gb[0].wait()
            gb[1].wait()
            _compute(col_blk.at[2 * j + 1], src_buf2, dst_buf2)

    plsc.subcore_barrier()

    for k in range((n_blocks + NUM_SUBCORES - 1) // NUM_SUBCORES):
        b = sid + k * NUM_SUBCORES

        @pl.when(b < n_blocks)
        def _():
            pltpu.sync_copy(shared.at[pl.ds(b * BLK, BLK)],
                            w_buf.at[pl.ds(0, BLK)])
            pltpu.sync_copy(w_buf.at[pl.ds(0, BLK)],
                            out.at[cid, pl.ds(b * BLK, BLK)])


def _sc_edge_pass(proj, rows, cols, att_flat):
    n = proj.shape[0]
    e = rows.shape[0] * rows.shape[1]
    mesh = plsc.VectorSubcoreMesh(core_axis_name="c", subcore_axis_name="s")
    return pl.kernel(
        functools.partial(_sc_edge_body, n, e),
        out_type=jax.ShapeDtypeStruct((NUM_CORES, n, PADW), jnp.float32),
        mesh=mesh,
        compiler_params=pltpu.CompilerParams(use_tc_tiling_on_sc=False,
                                             needs_layout_passes=False),
        scratch_types=[
            pltpu.VMEM((IBC, CHUNK), jnp.int32),
            pltpu.VMEM((IBC, CHUNK), jnp.int32),
            pltpu.VMEM((CHUNK, HC), jnp.float32),
            pltpu.VMEM((CHUNK, HC), jnp.float32),
            pltpu.VMEM((CHUNK, HC), jnp.float32),
            pltpu.VMEM((CHUNK, HC), jnp.float32),
            pltpu.VMEM((CHUNK, PADW), jnp.float32),
            pltpu.VMEM((HC,), jnp.float32),
            pltpu.VMEM_SHARED((n, PADW), jnp.float32),
            pltpu.SemaphoreType.DMA,
            pltpu.SemaphoreType.DMA,
        ],
    )(proj, rows, cols, att_flat)


def _finalize_body(p_ref, o_ref):
    p = p_ref[0] + p_ref[1]
    w = p[:, :HC]
    nrm = jnp.maximum(p[:, HC:HC + H], 1e-12)
    o_ref[...] = w / jnp.repeat(nrm, C, axis=1)


def _finalize(partials):
    n = partials.shape[1]
    bm = 1000
    return pl.pallas_call(
        _finalize_body,
        grid=(n // bm,),
        in_specs=[pl.BlockSpec((NUM_CORES, bm, PADW), lambda i: (0, i, 0))],
        out_specs=pl.BlockSpec((bm, HC), lambda i: (i, 0)),
        out_shape=jax.ShapeDtypeStruct((n, HC), jnp.float32),
    )(partials)


def kernel(x, edge_index, W, att):
    proj = _project(x, W)
    e = edge_index.shape[1]
    partials = _sc_edge_pass(proj,
                             edge_index[0].reshape(e // CHUNK, CHUNK),
                             edge_index[1].reshape(e // CHUNK, CHUNK),
                             att.reshape(-1))
    return _finalize(partials)
